# Initial kernel scaffold; baseline (speedup 1.0000x reference)
import jax
import jax.numpy as jnp
from jax.experimental import pallas as pl


def kernel(x, edge_index, W1, b1, W2, b2, W3, b3):
    raise NotImplementedError("write your pallas kernel here")


# speedup vs baseline: 6.3611x; 6.3611x over previous
import functools

import jax
import jax.numpy as jnp
from jax import lax
from jax.experimental import pallas as pl
from jax.experimental.pallas import tpu as pltpu
from jax.experimental.pallas import tpu_sc as plsc

N = 10000
E = 320000
D = 128
NC = 2
NS = 16
K = 128
CHUNKS = 80
EP = NC * NS * CHUNKS * K
RPS = 640
NR = NS * RPS
RB = 1024
GRID = NR // RB

_mesh = plsc.VectorSubcoreMesh(core_axis_name="core", subcore_axis_name="subcore")


def _sc_degree(dst4, onesD, zerosD):
    @functools.partial(
        pl.kernel,
        out_type=jax.ShapeDtypeStruct((NC, NR, D), jnp.float32),
        mesh=_mesh,
        scratch_types=[
            pltpu.VMEM((CHUNKS, K), jnp.int32),
            pltpu.VMEM((K, D), jnp.float32),
            pltpu.VMEM_SHARED((NR, D), jnp.float32),
        ],
    )
    def deg_kernel(dst_hbm, ones_hbm, zeros_hbm, out_hbm, didx, ones_v, acc):
        c = lax.axis_index("core")
        s = lax.axis_index("subcore")
        pltpu.sync_copy(zeros_hbm, acc.at[pl.ds(s * RPS, RPS)])
        pltpu.sync_copy(ones_hbm, ones_v)
        pltpu.sync_copy(dst_hbm.at[c, s], didx)
        plsc.subcore_barrier()

        @pl.loop(0, CHUNKS)
        def _(j):
            pltpu.sync_copy(ones_v, acc.at[didx.at[j]], add=True)

        plsc.subcore_barrier()
        pltpu.sync_copy(acc.at[pl.ds(s * RPS, RPS)],
                        out_hbm.at[c, pl.ds(s * RPS, RPS)])

    return deg_kernel(dst4, onesD, zerosD)


def _sc_segsum(g, src4, dst4, zerosD):
    @functools.partial(
        pl.kernel,
        out_type=jax.ShapeDtypeStruct((NC, NR, D), jnp.float32),
        mesh=_mesh,
        scratch_types=[
            pltpu.VMEM((CHUNKS, K), jnp.int32),
            pltpu.VMEM((CHUNKS, K), jnp.int32),
            pltpu.VMEM((K, D), jnp.float32),
            pltpu.VMEM_SHARED((NR, D), jnp.float32),
        ],
    )
    def seg_kernel(g_hbm, src_hbm, dst_hbm, zeros_hbm, out_hbm,
                   sidx, didx, rows, acc):
        c = lax.axis_index("core")
        s = lax.axis_index("subcore")
        pltpu.sync_copy(zeros_hbm, acc.at[pl.ds(s * RPS, RPS)])
        pltpu.sync_copy(src_hbm.at[c, s], sidx)
        pltpu.sync_copy(dst_hbm.at[c, s], didx)
        plsc.subcore_barrier()

        @pl.loop(0, CHUNKS)
        def _(j):
            pltpu.sync_copy(g_hbm.at[sidx.at[j]], rows)
            pltpu.sync_copy(rows, acc.at[didx.at[j]], add=True)

        plsc.subcore_barrier()
        pltpu.sync_copy(acc.at[pl.ds(s * RPS, RPS)],
                        out_hbm.at[c, pl.ds(s * RPS, RPS)])

    return seg_kernel(g, src4, dst4, zerosD)


def _first_body(x_ref, w_ref, deg_ref, g_ref, dinv_ref):
    d16 = deg_ref[0, :, 0:1] + deg_ref[1, :, 0:1]
    dinv = 1.0 / jnp.sqrt(d16 + 1.0)
    dinv_ref[...] = jnp.broadcast_to(dinv, (RB, D))
    g_ref[...] = jnp.dot(dinv * x_ref[...], w_ref[...],
                         preferred_element_type=jnp.float32)


def _tc_first(xp, W1, degp):
    return pl.pallas_call(
        _first_body,
        grid=(GRID,),
        in_specs=[
            pl.BlockSpec((RB, D), lambda i: (i, 0)),
            pl.BlockSpec((D, D), lambda i: (0, 0)),
            pl.BlockSpec((NC, RB, D), lambda i: (0, i, 0)),
        ],
        out_specs=[
            pl.BlockSpec((RB, D), lambda i: (i, 0)),
            pl.BlockSpec((RB, D), lambda i: (i, 0)),
        ],
        out_shape=[
            jax.ShapeDtypeStruct((NR, D), jnp.float32),
            jax.ShapeDtypeStruct((NR, D), jnp.float32),
        ],
    )(xp, W1, degp)


def _mid_body(p_ref, g_ref, dinv_ref, b_ref, w_ref, o_ref):
    dinv = dinv_ref[...]
    s = p_ref[0] + p_ref[1] + g_ref[...]
    a = jnp.maximum(dinv * s + b_ref[...], 0.0)
    o_ref[...] = jnp.dot(dinv * a, w_ref[...],
                         preferred_element_type=jnp.float32)


def _tc_mid(p, g, dinvb, b, Wn):
    return pl.pallas_call(
        _mid_body,
        grid=(GRID,),
        in_specs=[
            pl.BlockSpec((NC, RB, D), lambda i: (0, i, 0)),
            pl.BlockSpec((RB, D), lambda i: (i, 0)),
            pl.BlockSpec((RB, D), lambda i: (i, 0)),
            pl.BlockSpec((1, D), lambda i: (0, 0)),
            pl.BlockSpec((D, D), lambda i: (0, 0)),
        ],
        out_specs=pl.BlockSpec((RB, D), lambda i: (i, 0)),
        out_shape=jax.ShapeDtypeStruct((NR, D), jnp.float32),
    )(p, g, dinvb, b, Wn)


def _last_body(p_ref, g_ref, dinv_ref, b_ref, o_ref):
    s = p_ref[0] + p_ref[1] + g_ref[...]
    o_ref[...] = dinv_ref[...] * s + b_ref[...]


def _tc_last(p, g, dinvb, b):
    return pl.pallas_call(
        _last_body,
        grid=(GRID,),
        in_specs=[
            pl.BlockSpec((NC, RB, D), lambda i: (0, i, 0)),
            pl.BlockSpec((RB, D), lambda i: (i, 0)),
            pl.BlockSpec((RB, D), lambda i: (i, 0)),
            pl.BlockSpec((1, D), lambda i: (0, 0)),
        ],
        out_specs=pl.BlockSpec((RB, D), lambda i: (i, 0)),
        out_shape=jax.ShapeDtypeStruct((NR, D), jnp.float32),
    )(p, g, dinvb, b)


def kernel(x, edge_index, W1, b1, W2, b2, W3, b3):
    src = edge_index[0].astype(jnp.int32)
    dst = edge_index[1].astype(jnp.int32)
    pad = EP - E
    src4 = jnp.concatenate([src, jnp.zeros((pad,), jnp.int32)]).reshape(
        NC, NS, CHUNKS, K)
    dst4 = jnp.concatenate([dst, jnp.full((pad,), N, jnp.int32)]).reshape(
        NC, NS, CHUNKS, K)
    xp = jnp.zeros((NR, D), jnp.float32).at[:N].set(x)
    onesD = jnp.ones((K, D), jnp.float32)
    zerosD = jnp.zeros((RPS, D), jnp.float32)
    b1r, b2r, b3r = (b.reshape(1, D) for b in (b1, b2, b3))

    degp = _sc_degree(dst4, onesD, zerosD)
    g1, dinvb = _tc_first(xp, W1, degp)
    p1 = _sc_segsum(g1, src4, dst4, zerosD)
    g2 = _tc_mid(p1, g1, dinvb, b1r, W2)
    p2 = _sc_segsum(g2, src4, dst4, zerosD)
    g3 = _tc_mid(p2, g2, dinvb, b2r, W3)
    p3 = _sc_segsum(g3, src4, dst4, zerosD)
    out = _tc_last(p3, g3, dinvb, b3r)
    return out[:N]
